# Initial kernel scaffold; baseline (speedup 1.0000x reference)
import functools

import jax
import jax.numpy as jnp
from jax import lax
from jax.experimental import pallas as pl
from jax.experimental.pallas import tpu as pltpu
from jax.experimental.pallas import tpu_sc as plsc

_NC = 2
_NS = 16
_NW = _NC * _NS
_LANES = 16
_CHUNK = 80


def _sc_mesh():
    return plsc.VectorSubcoreMesh(core_axis_name="c", subcore_axis_name="s")


@functools.lru_cache(maxsize=None)
def _make_deg_kernel(e: int, npad: int):
    epw = e // _NW
    nch = epw // _CHUNK
    slab = npad // _NS
    zbuf = ((slab + _LANES - 1) // _LANES) * _LANES

    @functools.partial(
        pl.kernel,
        out_type=jax.ShapeDtypeStruct((_NC, npad), jnp.float32),
        mesh=_sc_mesh(),
        scratch_types=[
            pltpu.VMEM((_CHUNK,), jnp.int32),
            pltpu.VMEM((_CHUNK,), jnp.float32),
            pltpu.VMEM((zbuf,), jnp.float32),
            pltpu.VMEM_SHARED((npad,), jnp.float32),
        ],
    )
    def deg_kernel(src_hbm, out_hbm, sidx, ones_v, stage, acc):
        c = lax.axis_index("c")
        s = lax.axis_index("s")
        wid = s * _NC + c
        for j in range(_CHUNK // _LANES):
            ones_v[pl.ds(j * _LANES, _LANES)] = jnp.ones((_LANES,), jnp.float32)

        def zb(i, carry):
            stage[pl.ds(i * _LANES, _LANES)] = jnp.zeros((_LANES,), jnp.float32)
            return carry

        lax.fori_loop(0, zbuf // _LANES, zb, 0)
        pltpu.sync_copy(stage.at[pl.ds(0, slab)], acc.at[pl.ds(s * slab, slab)])
        plsc.subcore_barrier()

        def body(k, carry):
            base = wid * epw + k * _CHUNK
            pltpu.sync_copy(src_hbm.at[pl.ds(base, _CHUNK)], sidx)
            pltpu.sync_copy(ones_v, acc.at[sidx], add=True)
            return carry

        lax.fori_loop(0, nch, body, 0)
        plsc.subcore_barrier()
        pltpu.sync_copy(acc.at[pl.ds(s * slab, slab)], stage.at[pl.ds(0, slab)])
        pltpu.sync_copy(stage.at[pl.ds(0, slab)], out_hbm.at[c, pl.ds(s * slab, slab)])

    return deg_kernel


@functools.lru_cache(maxsize=None)
def _make_agg_kernel(e: int, npad: int, f: int):
    epw = e // _NW
    nch = epw // _CHUNK
    slab = npad // _NS
    zrows = slab // 8
    nvec = f // _LANES

    @functools.partial(
        pl.kernel,
        out_type=jax.ShapeDtypeStruct((_NC, npad, f), jnp.float32),
        mesh=_sc_mesh(),
        scratch_types=[
            pltpu.VMEM((_CHUNK,), jnp.int32),
            pltpu.VMEM((_CHUNK,), jnp.int32),
            pltpu.VMEM((_CHUNK, f), jnp.float32),
            pltpu.VMEM((zrows, f), jnp.float32),
            pltpu.VMEM_SHARED((npad, f), jnp.float32),
            pltpu.SemaphoreType.DMA,
        ],
    )
    def agg_kernel(x_hbm, src_hbm, dst_hbm, out_hbm, sidx, didx, rows, zstage, acc, sem):
        c = lax.axis_index("c")
        s = lax.axis_index("s")
        wid = s * _NC + c

        def zb(i, carry):
            for j in range(nvec):
                zstage[i, pl.ds(j * _LANES, _LANES)] = jnp.zeros((_LANES,), jnp.float32)
            return carry

        lax.fori_loop(0, zrows, zb, 0)
        for k in range(slab // zrows):
            pltpu.sync_copy(zstage, acc.at[pl.ds(s * slab + k * zrows, zrows)])
        plsc.subcore_barrier()

        def body(k, carry):
            base = wid * epw + k * _CHUNK
            pltpu.sync_copy(src_hbm.at[pl.ds(base, _CHUNK)], sidx)
            pltpu.sync_copy(dst_hbm.at[pl.ds(base, _CHUNK)], didx)
            pltpu.async_copy(x_hbm.at[sidx], rows, sem).wait()
            pltpu.sync_copy(rows, acc.at[didx], add=True)
            return carry

        lax.fori_loop(0, nch, body, 0)
        plsc.subcore_barrier()
        for k in range(slab // zrows):
            off = s * slab + k * zrows
            pltpu.sync_copy(acc.at[pl.ds(off, zrows)], zstage)
            pltpu.sync_copy(zstage, out_hbm.at[c, pl.ds(off, zrows)])

    return agg_kernel


def _tc_normalize(deg_parts3, feat_p):
    npad, f = feat_p.shape

    def body(deg_ref, feat_ref, xn_ref):
        deg = deg_ref[0] + deg_ref[1]
        rdeg = 1.0 / jnp.maximum(deg, 1.0)
        xn_ref[...] = feat_ref[...] * rdeg

    return pl.pallas_call(
        body, out_shape=jax.ShapeDtypeStruct((npad, f), jnp.float32)
    )(deg_parts3, feat_p)


def _tc_hidden(s1_parts, w1, b1r, deg_parts3):
    _, npad, f = s1_parts.shape
    hid = w1.shape[1]

    def body(s1_ref, w_ref, b_ref, deg_ref, hn_ref):
        s1 = s1_ref[0] + s1_ref[1]
        z = jnp.dot(s1, w_ref[...], preferred_element_type=jnp.float32)
        h = jnp.maximum(z + b_ref[...], 0.0)
        deg = deg_ref[0] + deg_ref[1]
        rdeg = 1.0 / jnp.maximum(deg, 1.0)
        hn_ref[...] = h * rdeg

    return pl.pallas_call(
        body, out_shape=jax.ShapeDtypeStruct((npad, hid), jnp.float32)
    )(s1_parts, w1, b1r, deg_parts3)


def _tc_final(g_parts, w_mu, b_mur, w_ls, b_lsr, noise_p):
    _, npad, _ = g_parts.shape
    out_f = w_mu.shape[1]

    def body(g_ref, wmu_ref, bmu_ref, wls_ref, bls_ref, noise_ref, out_ref):
        g = g_ref[0] + g_ref[1]
        mu = jnp.dot(g, wmu_ref[...], preferred_element_type=jnp.float32) + bmu_ref[...]
        ls = jnp.dot(g, wls_ref[...], preferred_element_type=jnp.float32) + bls_ref[...]
        out_ref[...] = mu + noise_ref[...] * jnp.exp(ls)

    return pl.pallas_call(
        body, out_shape=jax.ShapeDtypeStruct((npad, out_f), jnp.float32)
    )(g_parts, w_mu, b_mur, w_ls, b_lsr, noise_p)


def kernel(feat, edge_index, W1, b1, W_mu, b_mu, W_ls, b_ls):
    n, f = feat.shape
    e = edge_index.shape[1]
    npad = ((n + 127) // 128) * 128

    src = edge_index[0]
    dst = edge_index[1]
    feat_p = jnp.pad(feat, ((0, npad - n), (0, 0)))

    deg_parts = _make_deg_kernel(e, npad)(src)
    deg_parts3 = deg_parts.reshape(_NC, npad, 1)

    agg = _make_agg_kernel(e, npad, f)
    xn = _tc_normalize(deg_parts3, feat_p)
    s1_parts = agg(xn, src, dst)
    hn = _tc_hidden(s1_parts, W1, b1.reshape(1, -1), deg_parts3)
    g_parts = agg(hn, src, dst)

    noise = jax.random.normal(jax.random.key(42), (n, W_mu.shape[1]), dtype=jnp.float32)
    noise_p = jnp.pad(noise, ((0, npad - n), (0, 0)))
    out = _tc_final(g_parts, W_mu, b_mu.reshape(1, -1), W_ls, b_ls.reshape(1, -1), noise_p)
    return out[:n]


# speedup vs baseline: 4.2850x; 4.2850x over previous
import functools

import jax
import jax.numpy as jnp
from jax import lax
from jax.experimental import pallas as pl
from jax.experimental.pallas import tpu as pltpu
from jax.experimental.pallas import tpu_sc as plsc

_NC = 2
_NS = 16
_NW = _NC * _NS
_LANES = 16
_CHUNK = 80


def _sc_mesh():
    return plsc.VectorSubcoreMesh(core_axis_name="c", subcore_axis_name="s")


@functools.lru_cache(maxsize=None)
def _make_deg_kernel(e: int, npad: int):
    epw = e // _NW
    nch = epw // _CHUNK
    slab = npad // _NS
    zbuf = ((slab + _LANES - 1) // _LANES) * _LANES

    @functools.partial(
        pl.kernel,
        out_type=jax.ShapeDtypeStruct((_NC * npad,), jnp.float32),
        mesh=_sc_mesh(),
        scratch_types=[
            pltpu.VMEM((_CHUNK,), jnp.int32),
            pltpu.VMEM((_CHUNK,), jnp.float32),
            pltpu.VMEM((zbuf,), jnp.float32),
            pltpu.VMEM_SHARED((npad,), jnp.float32),
        ],
    )
    def deg_kernel(src_hbm, out_hbm, sidx, ones_v, stage, acc):
        c = lax.axis_index("c")
        s = lax.axis_index("s")
        wid = s * _NC + c
        for j in range(_CHUNK // _LANES):
            ones_v[pl.ds(j * _LANES, _LANES)] = jnp.ones((_LANES,), jnp.float32)

        def zb(i, carry):
            stage[pl.ds(i * _LANES, _LANES)] = jnp.zeros((_LANES,), jnp.float32)
            return carry

        lax.fori_loop(0, zbuf // _LANES, zb, 0)
        pltpu.sync_copy(stage.at[pl.ds(0, slab)], acc.at[pl.ds(s * slab, slab)])
        plsc.subcore_barrier()

        def body(k, carry):
            base = wid * epw + k * _CHUNK
            pltpu.sync_copy(src_hbm.at[pl.ds(base, _CHUNK)], sidx)
            pltpu.sync_copy(ones_v, acc.at[sidx], add=True)
            return carry

        lax.fori_loop(0, nch, body, 0)
        plsc.subcore_barrier()
        pltpu.sync_copy(acc.at[pl.ds(s * slab, slab)], stage.at[pl.ds(0, slab)])
        pltpu.sync_copy(stage.at[pl.ds(0, slab)], out_hbm.at[pl.ds(c * npad + s * slab, slab)])

    return deg_kernel


@functools.lru_cache(maxsize=None)
def _make_agg_kernel(e: int, npad: int, fh: int):
    ept = e // _NS
    nch = ept // _CHUNK
    slab = npad // _NS
    nvec = fh // _LANES

    @functools.partial(
        pl.kernel,
        out_type=jax.ShapeDtypeStruct((_NC * npad, fh), jnp.float32),
        mesh=_sc_mesh(),
        compiler_params=pltpu.CompilerParams(use_tc_tiling_on_sc=False),
        scratch_types=[
            pltpu.VMEM((_CHUNK,), jnp.int32),
            pltpu.VMEM((_CHUNK,), jnp.int32),
            pltpu.VMEM((_CHUNK, fh), jnp.float32),
            pltpu.VMEM((slab, fh), jnp.float32),
            pltpu.VMEM_SHARED((npad, fh), jnp.float32),
            pltpu.SemaphoreType.DMA,
        ],
    )
    def agg_kernel(x_hbm, src_hbm, dst_hbm, out_hbm, sidx, didx, rows, zstage, acc, sem):
        c = lax.axis_index("c")
        s = lax.axis_index("s")

        def zb(i, carry):
            for j in range(nvec):
                zstage[i, pl.ds(j * _LANES, _LANES)] = jnp.zeros((_LANES,), jnp.float32)
            return carry

        lax.fori_loop(0, slab, zb, 0)
        pltpu.sync_copy(zstage, acc.at[pl.ds(s * slab, slab)])
        plsc.subcore_barrier()

        row_off = c * npad

        def body(k, carry):
            base = s * ept + k * _CHUNK
            pltpu.sync_copy(src_hbm.at[pl.ds(base, _CHUNK)], sidx)
            pltpu.sync_copy(dst_hbm.at[pl.ds(base, _CHUNK)], didx)
            for j in range(_CHUNK // _LANES):
                sidx[pl.ds(j * _LANES, _LANES)] = sidx[pl.ds(j * _LANES, _LANES)] + row_off
            pltpu.async_copy(x_hbm.at[sidx], rows, sem).wait()
            pltpu.sync_copy(rows, acc.at[didx], add=True)
            return carry

        lax.fori_loop(0, nch, body, 0)
        plsc.subcore_barrier()
        pltpu.sync_copy(acc.at[pl.ds(s * slab, slab)], zstage)
        pltpu.sync_copy(zstage, out_hbm.at[pl.ds(row_off + s * slab, slab)])

    return agg_kernel


def _tc_normalize(deg_parts3, feat_p):
    npad, f = feat_p.shape
    fh = f // 2

    def body(deg_ref, feat_ref, xn_ref):
        deg = deg_ref[0] + deg_ref[1]
        rdeg = 1.0 / jnp.maximum(deg, 1.0)
        xf = feat_ref[...] * rdeg
        xn_ref[pl.ds(0, npad), :] = xf[:, :fh]
        xn_ref[pl.ds(npad, npad), :] = xf[:, fh:]

    return pl.pallas_call(
        body, out_shape=jax.ShapeDtypeStruct((2 * npad, fh), jnp.float32)
    )(deg_parts3, feat_p)


def _tc_hidden(s1_2, w1, b1r, deg_parts3):
    npad2, fh = s1_2.shape
    npad = npad2 // 2
    hid = w1.shape[1]
    hh = hid // 2

    def body(s1_ref, w_ref, b_ref, deg_ref, hn_ref):
        l = s1_ref[pl.ds(0, npad), :]
        r = s1_ref[pl.ds(npad, npad), :]
        z = (jnp.dot(l, w_ref[pl.ds(0, fh), :], preferred_element_type=jnp.float32)
             + jnp.dot(r, w_ref[pl.ds(fh, fh), :], preferred_element_type=jnp.float32)
             + b_ref[...])
        h = jnp.maximum(z, 0.0)
        deg = deg_ref[0] + deg_ref[1]
        rdeg = 1.0 / jnp.maximum(deg, 1.0)
        hs = h * rdeg
        hn_ref[pl.ds(0, npad), :] = hs[:, :hh]
        hn_ref[pl.ds(npad, npad), :] = hs[:, hh:]

    return pl.pallas_call(
        body, out_shape=jax.ShapeDtypeStruct((2 * npad, hh), jnp.float32)
    )(s1_2, w1, b1r, deg_parts3)


def _tc_final(g_2, w_mu, b_mur, w_ls, b_lsr, noise_p):
    npad2, fh = g_2.shape
    npad = npad2 // 2
    out_f = w_mu.shape[1]

    def body(g_ref, wmu_ref, bmu_ref, wls_ref, bls_ref, noise_ref, out_ref):
        l = g_ref[pl.ds(0, npad), :]
        r = g_ref[pl.ds(npad, npad), :]
        mu = (jnp.dot(l, wmu_ref[pl.ds(0, fh), :], preferred_element_type=jnp.float32)
              + jnp.dot(r, wmu_ref[pl.ds(fh, fh), :], preferred_element_type=jnp.float32)
              + bmu_ref[...])
        ls = (jnp.dot(l, wls_ref[pl.ds(0, fh), :], preferred_element_type=jnp.float32)
              + jnp.dot(r, wls_ref[pl.ds(fh, fh), :], preferred_element_type=jnp.float32)
              + bls_ref[...])
        out_ref[...] = mu + noise_ref[...] * jnp.exp(ls)

    return pl.pallas_call(
        body, out_shape=jax.ShapeDtypeStruct((npad, out_f), jnp.float32)
    )(g_2, w_mu, b_mur, w_ls, b_lsr, noise_p)


def kernel(feat, edge_index, W1, b1, W_mu, b_mu, W_ls, b_ls):
    n, f = feat.shape
    e = edge_index.shape[1]
    npad = ((n + 127) // 128) * 128

    src = edge_index[0]
    dst = edge_index[1]
    feat_p = jnp.pad(feat, ((0, npad - n), (0, 0)))

    deg_parts = _make_deg_kernel(e, npad)(src)
    deg_parts3 = deg_parts.reshape(_NC, npad, 1)

    agg = _make_agg_kernel(e, npad, f // 2)
    xn2 = _tc_normalize(deg_parts3, feat_p)
    s1_2 = agg(xn2, src, dst)
    hn2 = _tc_hidden(s1_2, W1, b1.reshape(1, -1), deg_parts3)
    g_2 = agg(hn2, src, dst)

    noise = jax.random.normal(jax.random.key(42), (n, W_mu.shape[1]), dtype=jnp.float32)
    noise_p = jnp.pad(noise, ((0, npad - n), (0, 0)))
    out = _tc_final(g_2, W_mu, b_mu.reshape(1, -1), W_ls, b_ls.reshape(1, -1), noise_p)
    return out[:n]


# speedup vs baseline: 10.2306x; 2.3876x over previous
import functools

import jax
import jax.numpy as jnp
from jax import lax
from jax.experimental import pallas as pl
from jax.experimental.pallas import tpu as pltpu
from jax.experimental.pallas import tpu_sc as plsc

_NC = 2
_NS = 16
_NW = _NC * _NS
_LANES = 16
_CHUNK = 80


def _sc_mesh():
    return plsc.VectorSubcoreMesh(core_axis_name="c", subcore_axis_name="s")


@functools.lru_cache(maxsize=None)
def _make_deg_kernel(e: int, npad: int):
    epw = e // _NW
    nch = epw // _CHUNK
    slab = npad // _NS
    zbuf = ((slab + _LANES - 1) // _LANES) * _LANES

    @functools.partial(
        pl.kernel,
        out_type=jax.ShapeDtypeStruct((_NC * npad,), jnp.float32),
        mesh=_sc_mesh(),
        scratch_types=[
            pltpu.VMEM((_CHUNK,), jnp.int32),
            pltpu.VMEM((_CHUNK,), jnp.float32),
            pltpu.VMEM((zbuf,), jnp.float32),
            pltpu.VMEM_SHARED((npad,), jnp.float32),
        ],
    )
    def deg_kernel(src_hbm, out_hbm, sidx, ones_v, stage, acc):
        c = lax.axis_index("c")
        s = lax.axis_index("s")
        wid = s * _NC + c
        for j in range(_CHUNK // _LANES):
            ones_v[pl.ds(j * _LANES, _LANES)] = jnp.ones((_LANES,), jnp.float32)

        def zb(i, carry):
            stage[pl.ds(i * _LANES, _LANES)] = jnp.zeros((_LANES,), jnp.float32)
            return carry

        lax.fori_loop(0, zbuf // _LANES, zb, 0)
        pltpu.sync_copy(stage.at[pl.ds(0, slab)], acc.at[pl.ds(s * slab, slab)])
        plsc.subcore_barrier()

        def body(k, carry):
            base = wid * epw + k * _CHUNK
            pltpu.sync_copy(src_hbm.at[pl.ds(base, _CHUNK)], sidx)
            pltpu.sync_copy(ones_v, acc.at[sidx], add=True)
            return carry

        lax.fori_loop(0, nch, body, 0)
        plsc.subcore_barrier()
        pltpu.sync_copy(acc.at[pl.ds(s * slab, slab)], stage.at[pl.ds(0, slab)])
        pltpu.sync_copy(stage.at[pl.ds(0, slab)], out_hbm.at[pl.ds(c * npad + s * slab, slab)])

    return deg_kernel


_K = 10
_SUPER = _K * _CHUNK


@functools.lru_cache(maxsize=None)
def _make_agg_kernel(e: int, npad: int, fh: int):
    ept = e // _NS
    nsc = ept // _SUPER
    slab = npad // _NS
    nvec = fh // _LANES

    @functools.partial(
        pl.kernel,
        out_type=jax.ShapeDtypeStruct((_NC * npad, fh), jnp.float32),
        mesh=_sc_mesh(),
        compiler_params=pltpu.CompilerParams(use_tc_tiling_on_sc=False),
        scratch_types=(
            [pltpu.VMEM((_CHUNK,), jnp.int32)] * (2 * _K)
            + [
                pltpu.VMEM((_SUPER, fh), jnp.float32),
                pltpu.VMEM_SHARED((npad, fh), jnp.float32),
                pltpu.SemaphoreType.DMA,
                pltpu.SemaphoreType.DMA,
                pltpu.SemaphoreType.DMA,
            ]
        ),
    )
    def agg_kernel(x_hbm, src_hbm, dst_hbm, out_hbm, *scr):
        sidx = scr[:_K]
        didx = scr[_K:2 * _K]
        rows, acc, sem_i, sem_g, sem_s = scr[2 * _K:]
        c = lax.axis_index("c")
        s = lax.axis_index("s")
        row_off = c * npad

        def zb(i, carry):
            for j in range(nvec):
                rows[i, pl.ds(j * _LANES, _LANES)] = jnp.zeros((_LANES,), jnp.float32)
            return carry

        lax.fori_loop(0, slab, zb, 0)
        pltpu.sync_copy(rows.at[pl.ds(0, slab)], acc.at[pl.ds(s * slab, slab)])
        plsc.subcore_barrier()

        def body(t, carry):
            base = s * ept + t * _SUPER
            cps = []
            for b in range(_K):
                cps.append(pltpu.async_copy(
                    src_hbm.at[pl.ds(base + b * _CHUNK, _CHUNK)], sidx[b], sem_i))
                cps.append(pltpu.async_copy(
                    dst_hbm.at[pl.ds(base + b * _CHUNK, _CHUNK)], didx[b], sem_i))
            for cp in cps:
                cp.wait()
            for b in range(_K):
                for j in range(_CHUNK // _LANES):
                    sidx[b][pl.ds(j * _LANES, _LANES)] = (
                        sidx[b][pl.ds(j * _LANES, _LANES)] + row_off)
            gs = [pltpu.async_copy(x_hbm.at[sidx[b]],
                                   rows.at[pl.ds(b * _CHUNK, _CHUNK)], sem_g)
                  for b in range(_K)]
            for cp in gs:
                cp.wait()
            ss = [pltpu.async_copy(rows.at[pl.ds(b * _CHUNK, _CHUNK)],
                                   acc.at[didx[b]], sem_s, add=True)
                  for b in range(_K)]
            for cp in ss:
                cp.wait()
            return carry

        lax.fori_loop(0, nsc, body, 0)
        plsc.subcore_barrier()
        pltpu.sync_copy(acc.at[pl.ds(s * slab, slab)], rows.at[pl.ds(0, slab)])
        pltpu.sync_copy(rows.at[pl.ds(0, slab)], out_hbm.at[pl.ds(row_off + s * slab, slab)])

    return agg_kernel


def _tc_normalize(deg_parts3, feat_p):
    npad, f = feat_p.shape
    fh = f // 2

    def body(deg_ref, feat_ref, xn_ref):
        deg = deg_ref[0] + deg_ref[1]
        rdeg = 1.0 / jnp.maximum(deg, 1.0)
        xf = feat_ref[...] * rdeg
        xn_ref[pl.ds(0, npad), :] = xf[:, :fh]
        xn_ref[pl.ds(npad, npad), :] = xf[:, fh:]

    return pl.pallas_call(
        body, out_shape=jax.ShapeDtypeStruct((2 * npad, fh), jnp.float32)
    )(deg_parts3, feat_p)


def _tc_hidden(s1_2, w1, b1r, deg_parts3):
    npad2, fh = s1_2.shape
    npad = npad2 // 2
    hid = w1.shape[1]
    hh = hid // 2

    def body(s1_ref, w_ref, b_ref, deg_ref, hn_ref):
        l = s1_ref[pl.ds(0, npad), :]
        r = s1_ref[pl.ds(npad, npad), :]
        z = (jnp.dot(l, w_ref[pl.ds(0, fh), :], preferred_element_type=jnp.float32)
             + jnp.dot(r, w_ref[pl.ds(fh, fh), :], preferred_element_type=jnp.float32)
             + b_ref[...])
        h = jnp.maximum(z, 0.0)
        deg = deg_ref[0] + deg_ref[1]
        rdeg = 1.0 / jnp.maximum(deg, 1.0)
        hs = h * rdeg
        hn_ref[pl.ds(0, npad), :] = hs[:, :hh]
        hn_ref[pl.ds(npad, npad), :] = hs[:, hh:]

    return pl.pallas_call(
        body, out_shape=jax.ShapeDtypeStruct((2 * npad, hh), jnp.float32)
    )(s1_2, w1, b1r, deg_parts3)


def _tc_final(g_2, w_mu, b_mur, w_ls, b_lsr, noise_p):
    npad2, fh = g_2.shape
    npad = npad2 // 2
    out_f = w_mu.shape[1]

    def body(g_ref, wmu_ref, bmu_ref, wls_ref, bls_ref, noise_ref, out_ref):
        l = g_ref[pl.ds(0, npad), :]
        r = g_ref[pl.ds(npad, npad), :]
        mu = (jnp.dot(l, wmu_ref[pl.ds(0, fh), :], preferred_element_type=jnp.float32)
              + jnp.dot(r, wmu_ref[pl.ds(fh, fh), :], preferred_element_type=jnp.float32)
              + bmu_ref[...])
        ls = (jnp.dot(l, wls_ref[pl.ds(0, fh), :], preferred_element_type=jnp.float32)
              + jnp.dot(r, wls_ref[pl.ds(fh, fh), :], preferred_element_type=jnp.float32)
              + bls_ref[...])
        out_ref[...] = mu + noise_ref[...] * jnp.exp(ls)

    return pl.pallas_call(
        body, out_shape=jax.ShapeDtypeStruct((npad, out_f), jnp.float32)
    )(g_2, w_mu, b_mur, w_ls, b_lsr, noise_p)


def kernel(feat, edge_index, W1, b1, W_mu, b_mu, W_ls, b_ls):
    n, f = feat.shape
    e = edge_index.shape[1]
    npad = ((n + 127) // 128) * 128

    src = edge_index[0]
    dst = edge_index[1]
    feat_p = jnp.pad(feat, ((0, npad - n), (0, 0)))

    deg_parts = _make_deg_kernel(e, npad)(src)
    deg_parts3 = deg_parts.reshape(_NC, npad, 1)

    agg = _make_agg_kernel(e, npad, f // 2)
    xn2 = _tc_normalize(deg_parts3, feat_p)
    s1_2 = agg(xn2, src, dst)
    hn2 = _tc_hidden(s1_2, W1, b1.reshape(1, -1), deg_parts3)
    g_2 = agg(hn2, src, dst)

    noise = jax.random.normal(jax.random.key(42), (n, W_mu.shape[1]), dtype=jnp.float32)
    noise_p = jnp.pad(noise, ((0, npad - n), (0, 0)))
    out = _tc_final(g_2, W_mu, b_mu.reshape(1, -1), W_ls, b_ls.reshape(1, -1), noise_p)
    return out[:n]
